# Initial kernel scaffold; baseline (speedup 1.0000x reference)
import jax
import jax.numpy as jnp
from jax.experimental import pallas as pl


def kernel(x_l, x_n, edge_l2n, edge_n2l, params):
    raise NotImplementedError("write your pallas kernel here")


# speedup vs baseline: 4.4898x; 4.4898x over previous
import functools

import jax
import jax.numpy as jnp
from jax import lax
from jax.experimental import pallas as pl
from jax.experimental.pallas import tpu as pltpu
from jax.experimental.pallas import tpu_sc as plsc

N = 10000
E = 320000
F = 128
NC = 2
NS = 16
NW = NC * NS
EPW = E // NW
CH = 80
NCH = EPW // CH
NP = 10240
RPT = NP // NS
DW = 8

_MESH = plsc.VectorSubcoreMesh(core_axis_name="c", subcore_axis_name="s")


def _sc_deg_body(edges_hbm, ones_hbm, zeros_hbm, out_hbm, idx_v, ones_v, acc):
    c = lax.axis_index("c")
    s = lax.axis_index("s")
    wid = s * NC + c
    pltpu.sync_copy(ones_hbm, ones_v)
    for a in range(4):
        pltpu.sync_copy(zeros_hbm.at[pl.ds(s * RPT, RPT)],
                        acc.at[pl.ds(s * RPT, RPT)])
        pltpu.sync_copy(edges_hbm.at[a, wid], idx_v)
        plsc.subcore_barrier()

        @pl.loop(0, NCH)
        def _chunk(j):
            pltpu.sync_copy(ones_v, acc.at[idx_v.at[j]], add=True)

        plsc.subcore_barrier()
        pltpu.sync_copy(acc.at[pl.ds(s * RPT, RPT)],
                        out_hbm.at[a, c, pl.ds(s * RPT, RPT)])
        plsc.subcore_barrier()


_sc_degrees = pl.kernel(
    _sc_deg_body,
    out_type=jax.ShapeDtypeStruct((4, NC, NP, F), jnp.float32),
    mesh=_MESH,
    scratch_types=[
        pltpu.VMEM((NCH, CH), jnp.int32),
        pltpu.VMEM((CH, F), jnp.float32),
        pltpu.VMEM_SHARED((NP, F), jnp.float32),
    ],
)


def _sc_conv_body(h_hbm, src_hbm, dst_hbm, zeros_hbm, out_hbm,
                  sidx_v, didx_v, rows_v, acc, sem):
    c = lax.axis_index("c")
    s = lax.axis_index("s")
    wid = s * NC + c
    pltpu.sync_copy(src_hbm.at[wid], sidx_v)
    pltpu.sync_copy(dst_hbm.at[wid], didx_v)
    pltpu.sync_copy(zeros_hbm.at[pl.ds(s * RPT, RPT)],
                    acc.at[pl.ds(s * RPT, RPT)])
    plsc.subcore_barrier()

    @pl.loop(0, NCH)
    def _chunk(j):
        pltpu.async_copy(h_hbm.at[sidx_v.at[j]], rows_v, sem).wait()
        pltpu.sync_copy(rows_v, acc.at[didx_v.at[j]], add=True)

    plsc.subcore_barrier()
    pltpu.sync_copy(acc.at[pl.ds(s * RPT, RPT)],
                    out_hbm.at[c, pl.ds(s * RPT, RPT)])


_sc_conv = pl.kernel(
    _sc_conv_body,
    out_type=jax.ShapeDtypeStruct((NC, NP, F), jnp.float32),
    mesh=_MESH,
    scratch_types=[
        pltpu.VMEM((NCH, CH), jnp.int32),
        pltpu.VMEM((NCH, CH), jnp.int32),
        pltpu.VMEM((CH, F), jnp.float32),
        pltpu.VMEM_SHARED((NP, F), jnp.float32),
        pltpu.SemaphoreType.DMA,
    ],
)


_BR = 2000


def _mm_body(x_ref, degp_ref, w_ref, o_ref):
    deg = degp_ref[0, :, 0:1] + degp_ref[1, :, 0:1]
    dinv = lax.rsqrt(jnp.maximum(deg, 1.0))
    o_ref[...] = jnp.dot(x_ref[...] * dinv, w_ref[...],
                         preferred_element_type=jnp.float32)


@functools.cache
def _make_mm(d):
    return pl.pallas_call(
        _mm_body,
        grid=(N // _BR,),
        in_specs=[
            pl.BlockSpec((_BR, d), lambda i: (i, 0)),
            pl.BlockSpec((NC, _BR, DW), lambda i: (0, i, 0)),
            pl.BlockSpec((d, F), lambda i: (0, 0)),
        ],
        out_specs=pl.BlockSpec((_BR, F), lambda i: (i, 0)),
        out_shape=jax.ShapeDtypeStruct((N, F), jnp.float32),
    )


def _post_body(p_ref, degp_ref, b_ref, g_ref, bb_ref, o_ref):
    x = p_ref[0] + p_ref[1]
    deg = degp_ref[0, :, 0:1] + degp_ref[1, :, 0:1]
    x = x * lax.rsqrt(jnp.maximum(deg, 1.0)) + b_ref[...]
    mu = jnp.mean(x, axis=-1, keepdims=True)
    var = jnp.mean((x - mu) ** 2, axis=-1, keepdims=True)
    y = (x - mu) * lax.rsqrt(var + 1e-5) * g_ref[...] + bb_ref[...]
    o_ref[...] = jnp.where(y > 0.0, y, jnp.exp(jnp.minimum(y, 0.0)) - 1.0)


_post = pl.pallas_call(
    _post_body,
    grid=(N // _BR,),
    in_specs=[
        pl.BlockSpec((NC, _BR, F), lambda i: (0, i, 0)),
        pl.BlockSpec((NC, _BR, DW), lambda i: (0, i, 0)),
        pl.BlockSpec((1, F), lambda i: (0, 0)),
        pl.BlockSpec((1, F), lambda i: (0, 0)),
        pl.BlockSpec((1, F), lambda i: (0, 0)),
    ],
    out_specs=pl.BlockSpec((_BR, F), lambda i: (i, 0)),
    out_shape=jax.ShapeDtypeStruct((N, F), jnp.float32),
)


def _readout_body(hn, hl, wm1, bm1, wm2, bm2, wd1, bd1, wd2, bd2, wd3, bd3,
                  wp1, bp1, wp2, bp2, wp3, bp3, rec_ref, prop_ref, z_ref):
    hg = jnp.mean(hn[...], axis=0, keepdims=True) \
        + jnp.mean(hl[...], axis=0, keepdims=True)
    hg8 = jnp.broadcast_to(hg, (8, F))

    def dot(a, b):
        return jnp.dot(a, b, preferred_element_type=jnp.float32)

    t = jnp.maximum(dot(hg8, wm1[...]) + bm1[...], 0.0)
    z = dot(t, wm2[...]) + bm2[...]
    d = jnp.maximum(dot(z, wd1[...]) + bd1[...], 0.0)
    d = jnp.maximum(dot(d, wd2[...]) + bd2[...], 0.0)
    rec = dot(d, wd3[...]) + bd3[...]
    p = jnp.maximum(dot(z, wp1[...]) + bp1[...], 0.0)
    p = jnp.maximum(dot(p, wp2[...]) + bp2[...], 0.0)
    prop = dot(p, wp3[...]) + bp3[...]
    rec_ref[...] = rec
    prop_ref[...] = prop
    z_ref[...] = z


_readout = pl.pallas_call(
    _readout_body,
    out_shape=(
        jax.ShapeDtypeStruct((8, 64), jnp.float32),
        jax.ShapeDtypeStruct((8, F), jnp.float32),
        jax.ShapeDtypeStruct((8, 64), jnp.float32),
    ),
)


def kernel(x_l, x_n, edge_l2n, edge_n2l, params):
    f32 = jnp.float32
    edges = jnp.concatenate(
        [edge_l2n.astype(jnp.int32), edge_n2l.astype(jnp.int32)], axis=0
    ).reshape(4, NW, NCH, CH)
    zeros_big = jnp.zeros((NP, F), f32)
    ones_deg = jnp.ones((CH, F), f32)

    degs = _sc_degrees(edges, ones_deg, zeros_big)[..., 0:DW]
    dp_sl, dp_dl, dp_sn, dp_dn = degs[0], degs[1], degs[2], degs[3]
    src_l2n, dst_l2n, src_n2l, dst_n2l = (
        edges[0], edges[1], edges[2], edges[3])

    def r1(v):
        return v.reshape(1, -1)

    h_l, h_n = x_l, x_n
    for i in range(3):
        mm_l = _make_mm(h_l.shape[1])
        mm_n = _make_mm(h_n.shape[1])
        hs_l2n = mm_l(h_l, dp_sl, params['W_l2n'][i])
        hs_n2l = mm_n(h_n, dp_sn, params['W_n2l'][i])
        agg_n = _sc_conv(hs_l2n, src_l2n, dst_l2n, zeros_big)
        agg_l = _sc_conv(hs_n2l, src_n2l, dst_n2l, zeros_big)
        h_n = _post(agg_n, dp_dl, r1(params['b_l2n'][i]),
                    r1(params['ln_g_n'][i]), r1(params['ln_b_n'][i]))
        h_l = _post(agg_l, dp_dn, r1(params['b_n2l'][i]),
                    r1(params['ln_g_l'][i]), r1(params['ln_b_l'][i]))

    wp3 = jnp.pad(params['Wp3'], ((0, 0), (0, F - 1)))
    bp3 = jnp.pad(r1(params['bp3']), ((0, 0), (0, F - 1)))
    rec8, prop8, z8 = _readout(
        h_n, h_l,
        params['Wm1'], r1(params['bm1']), params['Wm2'], r1(params['bm2']),
        params['Wd1'], r1(params['bd1']), params['Wd2'], r1(params['bd2']),
        params['Wd3'], r1(params['bd3']),
        params['Wp1'], r1(params['bp1']), params['Wp2'], r1(params['bp2']),
        wp3, bp3)
    return rec8[0:1, :], prop8[0:1, 0:1], z8[0:1, :]
